# Initial kernel scaffold; baseline (speedup 1.0000x reference)
import jax
import jax.numpy as jnp
from jax.experimental import pallas as pl


def kernel(x, phi_W1, phi_b1, phi_W2, phi_b2, q_W, q_b, k_W, k_b, xi_W1, xi_b1, xi_W2, xi_b2, rho_W1, rho_b1, rho_W2, rho_b2):
    raise NotImplementedError("write your pallas kernel here")


# speedup vs baseline: 11.9049x; 11.9049x over previous
import functools

import jax
import jax.numpy as jnp
from jax.experimental import pallas as pl
from jax.experimental.pallas import tpu as pltpu

B, L, D, H, O, TOPK = 8, 2048, 128, 128, 64, 64
_SCALE = float(H) ** 0.5
_BIG = 1 << 30


def _fused_body(x_ref, xt_ref, pw1, pb1, pw2, pb2, qw, qb, kw, kb,
                xw1, xb1, xw2, xb2, rw1, rb1, rw2, rb2, out_ref,
                s_ref, rm_ref, vals_ref, pairs_ref):
    x2 = x_ref[0]
    xt = xt_ref[0]

    colabs = jnp.sum(jnp.abs(xt), axis=0, keepdims=True)
    validc = colabs != 0.0
    validf = validc.astype(jnp.float32)
    count = jnp.sum(validf)
    rowabs = jnp.sum(jnp.abs(x2), axis=1, keepdims=True)
    validr = rowabs != 0.0

    h = jnp.maximum(jnp.dot(x2, pw1[...]) + pb1[...], 0.0)
    phi_x = jnp.dot(h, pw2[...]) + pb2[...]
    phi_pooled = jnp.dot(validf, phi_x) / jnp.maximum(count, 1.0)

    q = jnp.dot(x2, qw[...]) + qb[...]
    k = jnp.dot(x2, kw[...]) + kb[...]
    s = jax.lax.dot_general(q, k, (((1,), (1,)), ((), ())))
    s = s * (1.0 / _SCALE)
    ri = jax.lax.broadcasted_iota(jnp.int32, (L, L), 0)
    ci = jax.lax.broadcasted_iota(jnp.int32, (L, L), 1)
    ok = validr & validc & (ri != ci)
    sm = jnp.where(ok, s, -jnp.inf)
    s_ref[...] = sm.reshape(L // 8, 8, L)

    rm_ref[...] = jnp.max(sm, axis=1).reshape(16, 128)

    fi = (jax.lax.broadcasted_iota(jnp.int32, (16, 128), 0) * 128
          + jax.lax.broadcasted_iota(jnp.int32, (16, 128), 1))
    si8 = jax.lax.broadcasted_iota(jnp.int32, (8, L), 0)
    li8 = jax.lax.broadcasted_iota(jnp.int32, (8, L), 1)
    sv = jax.lax.broadcasted_iota(jnp.int32, (8, 128), 0)
    lv = jax.lax.broadcasted_iota(jnp.int32, (8, 128), 1)
    sp = jax.lax.broadcasted_iota(jnp.int32, (TOPK, 2 * D), 0)
    neg_inf = jnp.float32(-jnp.inf)

    def body(t, carry):
        rm = rm_ref[...]
        m = jnp.max(rm)
        r = jnp.min(jnp.where(rm == m, fi, _BIG))
        rb, sub = r // 8, r % 8
        blk = s_ref[pl.ds(rb, 1)][0]
        in_row = si8 == sub
        c = jnp.min(jnp.where(in_row & (blk == m), li8, _BIG))
        hit = in_row & (li8 == c)
        new_blk = jnp.where(hit, neg_inf, blk)
        s_ref[pl.ds(rb, 1)] = new_blk[None]
        new_rmax = jnp.max(jnp.where(in_row, new_blk, neg_inf))
        rm_ref[...] = jnp.where(fi == r, new_rmax, rm)
        vals_ref[...] = jnp.where((sv == 0) & (lv == t), m, vals_ref[...])
        xr = x_ref[0, pl.ds(r, 1), :]
        xc = x_ref[0, pl.ds(c, 1), :]
        pair_row = jnp.concatenate([xr, xc], axis=1)
        pairs_ref[...] = jnp.where(sp == t, pair_row, pairs_ref[...])
        return carry

    jax.lax.fori_loop(0, TOPK, body, 0)

    vals = vals_ref[0:1, 0:TOPK]
    mv = jnp.max(vals)
    e = jnp.exp(vals - mv)
    w = e / jnp.sum(e)

    pairs = pairs_ref[...]
    h1 = jnp.maximum(jnp.dot(pairs, xw1[...]) + xb1[...], 0.0)
    xi_x = jnp.dot(h1, xw2[...]) + xb2[...]
    xi_pooled = jnp.dot(w, xi_x)

    pooled = jnp.concatenate([phi_pooled, xi_pooled], axis=1)
    h2 = jnp.maximum(jnp.dot(pooled, rw1[...]) + rb1[...], 0.0)
    out_ref[0] = jnp.dot(h2, rw2[...]) + rb2[...]


def kernel(x, phi_W1, phi_b1, phi_W2, phi_b2, q_W, q_b, k_W, k_b,
           xi_W1, xi_b1, xi_W2, xi_b2, rho_W1, rho_b1, rho_W2, rho_b2):
    xt = jnp.swapaxes(x, 1, 2)

    def wspec(shape):
        return pl.BlockSpec(shape, lambda b: (0,) * len(shape))

    weights = [
        (phi_W1.T, wspec((D, H))), (phi_b1.reshape(1, H), wspec((1, H))),
        (phi_W2.T, wspec((H, H))), (phi_b2.reshape(1, H), wspec((1, H))),
        (q_W.T, wspec((D, H))), (q_b.reshape(1, H), wspec((1, H))),
        (k_W.T, wspec((D, H))), (k_b.reshape(1, H), wspec((1, H))),
        (xi_W1.T, wspec((2 * D, H))), (xi_b1.reshape(1, H), wspec((1, H))),
        (xi_W2.T, wspec((H, H))), (xi_b2.reshape(1, H), wspec((1, H))),
        (rho_W1.T, wspec((2 * H, H))), (rho_b1.reshape(1, H), wspec((1, H))),
        (rho_W2.T, wspec((H, O))), (rho_b2.reshape(1, O), wspec((1, O))),
    ]

    out = pl.pallas_call(
        _fused_body,
        grid=(B,),
        in_specs=[
            pl.BlockSpec((1, L, D), lambda b: (b, 0, 0)),
            pl.BlockSpec((1, D, L), lambda b: (b, 0, 0)),
        ] + [spec for _, spec in weights],
        out_specs=pl.BlockSpec((1, 1, O), lambda b: (b, 0, 0)),
        out_shape=jax.ShapeDtypeStruct((B, 1, O), jnp.float32),
        scratch_shapes=[
            pltpu.VMEM((L // 8, 8, L), jnp.float32),
            pltpu.VMEM((16, 128), jnp.float32),
            pltpu.VMEM((8, 128), jnp.float32),
            pltpu.VMEM((TOPK, 2 * D), jnp.float32),
        ],
        compiler_params=pltpu.CompilerParams(
            dimension_semantics=("arbitrary",),
        ),
    )(x, xt, *[w for w, _ in weights])
    return out.reshape(B, O)
